# baseline (speedup 1.0000x reference)
import jax
import jax.numpy as jnp
from jax import lax
from jax.experimental import pallas as pl
from jax.experimental.pallas import tpu as pltpu
from jax.experimental.pallas import tpu_sc as plsc

NC = 2
NS = 16
NW = NC * NS

N = 10000
NP = 10240
E = 320000
EP = 322560
D = 128
V = 1000

CHUNK = 80
EPT = EP // NW
NCHUNKS = EPT // CHUNK
NPAIRS = NCHUNKS // 2 - 1
RPT = NP // NS
GPT = NP // NW
GCH = 40
NCPAD = 6
DEGW = 128


def _mesh():
    return plsc.VectorSubcoreMesh(
        core_axis_name="c", subcore_axis_name="s", num_cores=NC, num_subcores=NS
    )


def _emb_body(emb_h, x_h,
              h_out,
              xidx_v, rows_v, gsem):
    cid = lax.axis_index("c")
    sid = lax.axis_index("s")
    wid = sid * NC + cid

    pltpu.sync_copy(x_h.at[pl.ds(wid * (GPT // GCH), GPT // GCH)], xidx_v)
    for j in range(GPT // GCH):
        pltpu.async_copy(emb_h.at[xidx_v.at[j]],
                         rows_v.at[pl.ds(j * GCH, GCH)], gsem)
    for j in range(GPT // GCH):
        pltpu.make_async_copy(emb_h.at[xidx_v.at[0]],
                              rows_v.at[pl.ds(j * GCH, GCH)], gsem).wait()
    pltpu.sync_copy(rows_v, h_out.at[pl.ds(wid * GPT, GPT)])


def _seg_sum_body(h_h, src_h, dst_h, z_h,
                  p_out,
                  sidx_a, sidx_b, didx_a, didx_b, rows_a, rows_b, z_v,
                  semi_a, semi_b, sem_a, sem_b, agg_sh):
    cid = lax.axis_index("c")
    sid = lax.axis_index("s")
    wid = sid * NC + cid
    ebase = wid * EPT

    pltpu.sync_copy(z_h, z_v)
    for k in range(RPT // 64):
        pltpu.sync_copy(z_v, agg_sh.at[pl.ds(sid * RPT + k * 64, 64)])
    plsc.subcore_barrier()

    def pair_step(i, carry):
        base = pl.multiple_of(ebase + i * (2 * CHUNK), 8)
        pltpu.sync_copy(src_h.at[pl.ds(base, CHUNK)], sidx_a)
        pltpu.sync_copy(src_h.at[pl.ds(base + CHUNK, CHUNK)], sidx_b)
        da = pltpu.async_copy(h_h.at[sidx_a], rows_a, sem_a)
        db = pltpu.async_copy(h_h.at[sidx_b], rows_b, sem_b)
        pltpu.sync_copy(dst_h.at[pl.ds(base, CHUNK)], didx_a)
        pltpu.sync_copy(dst_h.at[pl.ds(base + CHUNK, CHUNK)], didx_b)
        da.wait()
        pltpu.sync_copy(rows_a, agg_sh.at[didx_a], add=True)
        db.wait()
        pltpu.sync_copy(rows_b, agg_sh.at[didx_b], add=True)
        return carry

    lax.fori_loop(0, NCHUNKS // 2, pair_step, 0)
    plsc.subcore_barrier()

    pltpu.sync_copy(agg_sh.at[pl.ds(sid * RPT, RPT)],
                    p_out.at[pl.ds(cid * NP + sid * RPT, RPT)])


def _layer_body(p_ref, deg_ref, h_ref, wl_ref, wr_ref, bl_ref, o_ref):
    deg = deg_ref[0] + deg_ref[1]
    agg = (p_ref[0] + p_ref[1]) / jnp.maximum(deg, 1.0)
    t = jnp.dot(agg, wl_ref[...], preferred_element_type=jnp.float32)
    t = t + jnp.dot(h_ref[...], wr_ref[...], preferred_element_type=jnp.float32)
    o_ref[...] = jnp.maximum(t + bl_ref[...], 0.0)


def _layer_tc(p, deg, h, wlT, wrT, bl):
    bn = 1024
    grid = (NP // bn,)
    return pl.pallas_call(
        _layer_body,
        grid=grid,
        in_specs=[
            pl.BlockSpec((NC, bn, D), lambda i: (0, i, 0)),
            pl.BlockSpec((NC, bn, 1), lambda i: (0, i, 0)),
            pl.BlockSpec((bn, D), lambda i: (i, 0)),
            pl.BlockSpec((D, D), lambda i: (0, 0)),
            pl.BlockSpec((D, D), lambda i: (0, 0)),
            pl.BlockSpec((1, D), lambda i: (0, 0)),
        ],
        out_specs=pl.BlockSpec((bn, D), lambda i: (i, 0)),
        out_shape=jax.ShapeDtypeStruct((NP, D), jnp.float32),
    )(p, deg, h, wlT, wrT, bl)


def _final_body(h_ref, wt_ref, b_ref, o_ref):
    logits = jnp.dot(h_ref[...], wt_ref[...],
                     preferred_element_type=jnp.float32) + b_ref[...]
    m = jnp.max(logits, axis=1, keepdims=True)
    e = jnp.exp(logits - m)
    o_ref[...] = e / jnp.sum(e, axis=1, keepdims=True)


def _final_tc(h, wT, b):
    bn = 1000
    grid = (N // bn,)
    return pl.pallas_call(
        _final_body,
        grid=grid,
        in_specs=[
            pl.BlockSpec((bn, D), lambda i: (i, 0)),
            pl.BlockSpec((D, V), lambda i: (0, 0)),
            pl.BlockSpec((1, V), lambda i: (0, 0)),
        ],
        out_specs=pl.BlockSpec((bn, V), lambda i: (i, 0)),
        out_shape=jax.ShapeDtypeStruct((N, V), jnp.float32),
    )(h, wT, b)


def kernel(x, edge_index, emb, Wl0, bl0, Wr0, Wl1, bl1, Wr1, Wl2, bl2, Wr2,
           W_last, b_last):
    src = edge_index[0]
    dst = edge_index[1]
    pad = EP - E
    src2 = jnp.concatenate([src, jnp.zeros((pad,), src.dtype)])
    dst2 = jnp.concatenate([dst, jnp.full((pad,), N, dst.dtype)])
    x2 = jnp.concatenate([x, jnp.zeros((NP - N,), x.dtype)]).reshape(
        NP // GCH, GCH)

    z128 = jnp.zeros((64, D), jnp.float32)

    emb_gather = pl.kernel(
        _emb_body,
        out_type=jax.ShapeDtypeStruct((NP, D), jnp.float32),
        mesh=_mesh(),
        scratch_types=[
            pltpu.VMEM((GPT // GCH, GCH), jnp.int32),
            pltpu.VMEM((GPT, D), jnp.float32),
            pltpu.SemaphoreType.DMA,
        ],
    )
    h = emb_gather(emb, x2)

    seg_sum = pl.kernel(
        _seg_sum_body,
        out_type=jax.ShapeDtypeStruct((NC * NP, D), jnp.float32),
        mesh=_mesh(),
        scratch_types=[
            pltpu.VMEM((CHUNK,), jnp.int32),
            pltpu.VMEM((CHUNK,), jnp.int32),
            pltpu.VMEM((CHUNK,), jnp.int32),
            pltpu.VMEM((CHUNK,), jnp.int32),
            pltpu.VMEM((CHUNK, D), jnp.float32),
            pltpu.VMEM((CHUNK, D), jnp.float32),
            pltpu.VMEM((64, D), jnp.float32),
            pltpu.SemaphoreType.DMA,
            pltpu.SemaphoreType.DMA,
            pltpu.SemaphoreType.DMA,
            pltpu.SemaphoreType.DMA,
            pltpu.VMEM_SHARED((NP, D), jnp.float32),
        ],
    )

    ones_nd = h * 0.0 + 1.0
    zsrc2 = src2 * 0
    deg_raw = seg_sum(ones_nd, zsrc2, dst2, z128)
    deg = deg_raw.reshape(NC, NP, D)[:, :, 0:1]
    h, _ = lax.optimization_barrier((h, deg_raw))

    for Wl, bl, Wr in ((Wl0, bl0, Wr0), (Wl1, bl1, Wr1), (Wl2, bl2, Wr2)):
        p = seg_sum(h, src2, dst2, z128).reshape(NC, NP, D)
        h = _layer_tc(p, deg, h, Wl.T, Wr.T, bl.reshape(1, D))

    return _final_tc(h, W_last.T, b_last.reshape(1, V))


# speedup vs baseline: 9.5577x; 9.5577x over previous
import jax
import jax.numpy as jnp
from jax import lax
from jax.experimental import pallas as pl
from jax.experimental.pallas import tpu as pltpu
from jax.experimental.pallas import tpu_sc as plsc

NC = 2
NS = 16
NW = NC * NS

N = 10000
NP = 10240
E = 320000
EP = 322560
D = 128
V = 1000

CHUNK = 80
EPT = EP // NW
NCHUNKS = EPT // CHUNK
NPAIRS = NCHUNKS // 2 - 1
RPT = NP // NS
GPT = NP // NW
GCH = 40
NCPAD = 6
DEGW = 128


def _mesh():
    return plsc.VectorSubcoreMesh(
        core_axis_name="c", subcore_axis_name="s", num_cores=NC, num_subcores=NS
    )


def _emb_body(emb_h, x_h,
              h_out,
              xidx_v, rows_v, gsem):
    cid = lax.axis_index("c")
    sid = lax.axis_index("s")
    wid = sid * NC + cid

    pltpu.sync_copy(x_h.at[pl.ds(wid * (GPT // GCH), GPT // GCH)], xidx_v)
    for j in range(GPT // GCH):
        pltpu.async_copy(emb_h.at[xidx_v.at[j]],
                         rows_v.at[pl.ds(j * GCH, GCH)], gsem)
    for j in range(GPT // GCH):
        pltpu.make_async_copy(emb_h.at[xidx_v.at[0]],
                              rows_v.at[pl.ds(j * GCH, GCH)], gsem).wait()
    pltpu.sync_copy(rows_v, h_out.at[pl.ds(wid * GPT, GPT)])


def _seg_sum_body(h_h, src_h, dst_h, z_h,
                  p_out,
                  sidx_a, sidx_b, didx_a, didx_b, rows_a, rows_b, z_v,
                  semi_a, semi_b, sem_a, sem_b, agg_sh):
    cid = lax.axis_index("c")
    sid = lax.axis_index("s")
    wid = sid * NC + cid
    ebase = wid * EPT

    pltpu.sync_copy(z_h, z_v)
    for k in range(RPT // 64):
        pltpu.sync_copy(z_v, agg_sh.at[pl.ds(sid * RPT + k * 64, 64)])
    plsc.subcore_barrier()

    def pair_step(i, carry):
        base = pl.multiple_of(ebase + i * (2 * CHUNK), 8)
        pltpu.sync_copy(src_h.at[pl.ds(base, CHUNK)], sidx_a)
        pltpu.sync_copy(src_h.at[pl.ds(base + CHUNK, CHUNK)], sidx_b)
        da = pltpu.async_copy(h_h.at[sidx_a], rows_a, sem_a)
        db = pltpu.async_copy(h_h.at[sidx_b], rows_b, sem_b)
        pltpu.sync_copy(dst_h.at[pl.ds(base, CHUNK)], didx_a)
        pltpu.sync_copy(dst_h.at[pl.ds(base + CHUNK, CHUNK)], didx_b)
        da.wait()
        pltpu.sync_copy(rows_a, agg_sh.at[didx_a], add=True)
        db.wait()
        pltpu.sync_copy(rows_b, agg_sh.at[didx_b], add=True)
        return carry

    lax.fori_loop(0, NCHUNKS // 2, pair_step, 0)
    plsc.subcore_barrier()

    pltpu.sync_copy(agg_sh.at[pl.ds(sid * RPT, RPT)],
                    p_out.at[pl.ds(cid * NP + sid * RPT, RPT)])


def _layer_body(p_ref, deg_ref, h_ref, wl_ref, wr_ref, bl_ref, o_ref):
    deg = deg_ref[0] + deg_ref[1]
    agg = (p_ref[0] + p_ref[1]) / jnp.maximum(deg, 1.0)
    t = jnp.dot(agg, wl_ref[...], preferred_element_type=jnp.float32)
    t = t + jnp.dot(h_ref[...], wr_ref[...], preferred_element_type=jnp.float32)
    o_ref[...] = jnp.maximum(t + bl_ref[...], 0.0)


def _layer_tc(p, deg, h, wlT, wrT, bl):
    bn = 1024
    grid = (NP // bn,)
    return pl.pallas_call(
        _layer_body,
        grid=grid,
        in_specs=[
            pl.BlockSpec((NC, bn, D), lambda i: (0, i, 0)),
            pl.BlockSpec((NC, bn, 1), lambda i: (0, i, 0)),
            pl.BlockSpec((bn, D), lambda i: (i, 0)),
            pl.BlockSpec((D, D), lambda i: (0, 0)),
            pl.BlockSpec((D, D), lambda i: (0, 0)),
            pl.BlockSpec((1, D), lambda i: (0, 0)),
        ],
        out_specs=pl.BlockSpec((bn, D), lambda i: (i, 0)),
        out_shape=jax.ShapeDtypeStruct((NP, D), jnp.float32),
    )(p, deg, h, wlT, wrT, bl)


def _final_body(h_ref, wt_ref, b_ref, o_ref):
    logits = jnp.dot(h_ref[...], wt_ref[...],
                     preferred_element_type=jnp.float32) + b_ref[...]
    m = jnp.max(logits, axis=1, keepdims=True)
    e = jnp.exp(logits - m)
    o_ref[...] = e / jnp.sum(e, axis=1, keepdims=True)


def _final_tc(h, wT, b):
    bn = 1000
    grid = (N // bn,)
    return pl.pallas_call(
        _final_body,
        grid=grid,
        in_specs=[
            pl.BlockSpec((bn, D), lambda i: (i, 0)),
            pl.BlockSpec((D, V), lambda i: (0, 0)),
            pl.BlockSpec((1, V), lambda i: (0, 0)),
        ],
        out_specs=pl.BlockSpec((bn, V), lambda i: (i, 0)),
        out_shape=jax.ShapeDtypeStruct((N, V), jnp.float32),
    )(h, wT, b)


def kernel(x, edge_index, emb, Wl0, bl0, Wr0, Wl1, bl1, Wr1, Wl2, bl2, Wr2,
           W_last, b_last):
    src = edge_index[0]
    dst = edge_index[1]
    pad = EP - E
    src2 = jnp.concatenate([src, jnp.zeros((pad,), src.dtype)])
    dst2 = jnp.concatenate([dst, jnp.full((pad,), N, dst.dtype)])
    x2 = jnp.concatenate([x, jnp.zeros((NP - N,), x.dtype)]).reshape(
        NP // GCH, GCH)

    z128 = jnp.zeros((64, D), jnp.float32)

    emb_gather = pl.kernel(
        _emb_body,
        out_type=jax.ShapeDtypeStruct((NP, D), jnp.float32),
        mesh=_mesh(),
        scratch_types=[
            pltpu.VMEM((GPT // GCH, GCH), jnp.int32),
            pltpu.VMEM((GPT, D), jnp.float32),
            pltpu.SemaphoreType.DMA,
        ],
    )
    h = emb_gather(emb, x2)

    seg_sum = pl.kernel(
        _seg_sum_body,
        out_type=jax.ShapeDtypeStruct((NC * NP, D), jnp.float32),
        mesh=_mesh(),
        scratch_types=[
            pltpu.VMEM((CHUNK,), jnp.int32),
            pltpu.VMEM((CHUNK,), jnp.int32),
            pltpu.VMEM((CHUNK,), jnp.int32),
            pltpu.VMEM((CHUNK,), jnp.int32),
            pltpu.VMEM((CHUNK, D), jnp.float32),
            pltpu.VMEM((CHUNK, D), jnp.float32),
            pltpu.VMEM((64, D), jnp.float32),
            pltpu.SemaphoreType.DMA,
            pltpu.SemaphoreType.DMA,
            pltpu.SemaphoreType.DMA,
            pltpu.SemaphoreType.DMA,
            pltpu.VMEM_SHARED((NP, D), jnp.float32),
        ],
    )

    ones_nd = h * 0.0 + 1.0
    deg_raw = seg_sum(ones_nd, src2, dst2, z128)
    deg = deg_raw.reshape(NC, NP, D)[:, :, 0:1]
    h, _ = lax.optimization_barrier((h, deg_raw))

    for Wl, bl, Wr in ((Wl0, bl0, Wr0), (Wl1, bl1, Wr1), (Wl2, bl2, Wr2)):
        p = seg_sum(h, src2, dst2, z128).reshape(NC, NP, D)
        h = _layer_tc(p, deg, h, Wl.T, Wr.T, bl.reshape(1, D))

    return _final_tc(h, W_last.T, b_last.reshape(1, V))


# speedup vs baseline: 11.6199x; 1.2158x over previous
import jax
import jax.numpy as jnp
from jax import lax
from jax.experimental import pallas as pl
from jax.experimental.pallas import tpu as pltpu
from jax.experimental.pallas import tpu_sc as plsc

NC = 2
NS = 16
NW = NC * NS

N = 10000
NP = 10240
E = 320000
EP = 322560
D = 128
V = 1000

CHUNK = 80
EPT = EP // NW
NCHUNKS = EPT // CHUNK
NBUF = 3
RPT = NP // NS
GPT = NP // NW
GCH = 40
NCPAD = 6
DEGW = 128


def _mesh():
    return plsc.VectorSubcoreMesh(
        core_axis_name="c", subcore_axis_name="s", num_cores=NC, num_subcores=NS
    )


def _emb_body(emb_h, x_h,
              h_out,
              xidx_v, rows_v, gsem):
    cid = lax.axis_index("c")
    sid = lax.axis_index("s")
    wid = sid * NC + cid

    pltpu.sync_copy(x_h.at[pl.ds(wid * (GPT // GCH), GPT // GCH)], xidx_v)
    for j in range(GPT // GCH):
        pltpu.async_copy(emb_h.at[xidx_v.at[j]],
                         rows_v.at[pl.ds(j * GCH, GCH)], gsem)
    for j in range(GPT // GCH):
        pltpu.make_async_copy(emb_h.at[xidx_v.at[0]],
                              rows_v.at[pl.ds(j * GCH, GCH)], gsem).wait()
    pltpu.sync_copy(rows_v, h_out.at[pl.ds(wid * GPT, GPT)])


def _seg_sum_body(h_h, src_h, dst_h, z_h,
                  p_out,
                  sidx0, sidx1, sidx2, didx0, didx1, didx2,
                  rows0, rows1, rows2, z_v, semg, semi, agg_sh):
    cid = lax.axis_index("c")
    sid = lax.axis_index("s")
    wid = sid * NC + cid
    ebase = wid * EPT
    sidx = [sidx0, sidx1, sidx2]
    didx = [didx0, didx1, didx2]
    rows = [rows0, rows1, rows2]

    pltpu.sync_copy(z_h, z_v)
    for k in range(RPT // 64):
        pltpu.sync_copy(z_v, agg_sh.at[pl.ds(sid * RPT + k * 64, 64)])
    plsc.subcore_barrier()

    def group_step(i, carry):
        base = pl.multiple_of(ebase + i * (NBUF * CHUNK), 8)
        idone = []
        for j in range(NBUF):
            idone.append(pltpu.async_copy(
                src_h.at[pl.ds(base + j * CHUNK, CHUNK)], sidx[j], semi))
            idone.append(pltpu.async_copy(
                dst_h.at[pl.ds(base + j * CHUNK, CHUNK)], didx[j], semi))
        for d in idone:
            d.wait()
        gd = [pltpu.async_copy(h_h.at[sidx[j]], rows[j], semg)
              for j in range(NBUF)]
        for j in range(NBUF):
            gd[j].wait()
            pltpu.sync_copy(rows[j], agg_sh.at[didx[j]], add=True)
        return carry

    lax.fori_loop(0, NCHUNKS // NBUF, group_step, 0)
    plsc.subcore_barrier()

    pltpu.sync_copy(agg_sh.at[pl.ds(sid * RPT, RPT)],
                    p_out.at[pl.ds(cid * NP + sid * RPT, RPT)])


def _layer_body(p_ref, deg_ref, h_ref, wl_ref, wr_ref, bl_ref, o_ref):
    deg = deg_ref[0] + deg_ref[1]
    agg = (p_ref[0] + p_ref[1]) / jnp.maximum(deg, 1.0)
    t = jnp.dot(agg, wl_ref[...], preferred_element_type=jnp.float32)
    t = t + jnp.dot(h_ref[...], wr_ref[...], preferred_element_type=jnp.float32)
    o_ref[...] = jnp.maximum(t + bl_ref[...], 0.0)


def _layer_tc(p, deg, h, wlT, wrT, bl):
    bn = 1024
    grid = (NP // bn,)
    return pl.pallas_call(
        _layer_body,
        grid=grid,
        in_specs=[
            pl.BlockSpec((NC, bn, D), lambda i: (0, i, 0)),
            pl.BlockSpec((NC, bn, 1), lambda i: (0, i, 0)),
            pl.BlockSpec((bn, D), lambda i: (i, 0)),
            pl.BlockSpec((D, D), lambda i: (0, 0)),
            pl.BlockSpec((D, D), lambda i: (0, 0)),
            pl.BlockSpec((1, D), lambda i: (0, 0)),
        ],
        out_specs=pl.BlockSpec((bn, D), lambda i: (i, 0)),
        out_shape=jax.ShapeDtypeStruct((NP, D), jnp.float32),
    )(p, deg, h, wlT, wrT, bl)


def _final_body(h_ref, wt_ref, b_ref, o_ref):
    logits = jnp.dot(h_ref[...], wt_ref[...],
                     preferred_element_type=jnp.float32) + b_ref[...]
    m = jnp.max(logits, axis=1, keepdims=True)
    e = jnp.exp(logits - m)
    o_ref[...] = e / jnp.sum(e, axis=1, keepdims=True)


def _final_tc(h, wT, b):
    bn = 1000
    grid = (N // bn,)
    return pl.pallas_call(
        _final_body,
        grid=grid,
        in_specs=[
            pl.BlockSpec((bn, D), lambda i: (i, 0)),
            pl.BlockSpec((D, V), lambda i: (0, 0)),
            pl.BlockSpec((1, V), lambda i: (0, 0)),
        ],
        out_specs=pl.BlockSpec((bn, V), lambda i: (i, 0)),
        out_shape=jax.ShapeDtypeStruct((N, V), jnp.float32),
    )(h, wT, b)


def kernel(x, edge_index, emb, Wl0, bl0, Wr0, Wl1, bl1, Wr1, Wl2, bl2, Wr2,
           W_last, b_last):
    src = edge_index[0]
    dst = edge_index[1]
    pad = EP - E
    src2 = jnp.concatenate([src, jnp.zeros((pad,), src.dtype)])
    dst2 = jnp.concatenate([dst, jnp.full((pad,), N, dst.dtype)])
    x2 = jnp.concatenate([x, jnp.zeros((NP - N,), x.dtype)]).reshape(
        NP // GCH, GCH)

    z128 = jnp.zeros((64, D), jnp.float32)

    emb_gather = pl.kernel(
        _emb_body,
        out_type=jax.ShapeDtypeStruct((NP, D), jnp.float32),
        mesh=_mesh(),
        scratch_types=[
            pltpu.VMEM((GPT // GCH, GCH), jnp.int32),
            pltpu.VMEM((GPT, D), jnp.float32),
            pltpu.SemaphoreType.DMA,
        ],
    )
    h = emb_gather(emb, x2)

    seg_sum = pl.kernel(
        _seg_sum_body,
        out_type=jax.ShapeDtypeStruct((NC * NP, D), jnp.float32),
        mesh=_mesh(),
        scratch_types=[
            pltpu.VMEM((CHUNK,), jnp.int32),
            pltpu.VMEM((CHUNK,), jnp.int32),
            pltpu.VMEM((CHUNK,), jnp.int32),
            pltpu.VMEM((CHUNK,), jnp.int32),
            pltpu.VMEM((CHUNK,), jnp.int32),
            pltpu.VMEM((CHUNK,), jnp.int32),
            pltpu.VMEM((CHUNK, D), jnp.float32),
            pltpu.VMEM((CHUNK, D), jnp.float32),
            pltpu.VMEM((CHUNK, D), jnp.float32),
            pltpu.VMEM((64, D), jnp.float32),
            pltpu.SemaphoreType.DMA,
            pltpu.SemaphoreType.DMA,
            pltpu.VMEM_SHARED((NP, D), jnp.float32),
        ],
    )

    ones_nd = h * 0.0 + 1.0
    deg_raw = seg_sum(ones_nd, src2, dst2, z128)
    deg = deg_raw.reshape(NC, NP, D)[:, :, 0:1]
    h, _ = lax.optimization_barrier((h, deg_raw))

    for Wl, bl, Wr in ((Wl0, bl0, Wr0), (Wl1, bl1, Wr1), (Wl2, bl2, Wr2)):
        p = seg_sum(h, src2, dst2, z128).reshape(NC, NP, D)
        h = _layer_tc(p, deg, h, Wl.T, Wr.T, bl.reshape(1, D))

    return _final_tc(h, W_last.T, b_last.reshape(1, V))
